# baseline (speedup 1.0000x reference)
import functools

import jax
import jax.numpy as jnp
from jax.experimental import pallas as pl
from jax.experimental.pallas import tpu as pltpu

EPS_ = 1e-6


def _stats_kernel(p_ref, g_ref, label_ref, out_t_ref, out_v_ref, *, C):
    b = pl.program_id(0)
    p = p_ref[0]
    g = g_ref[0]
    pos = jnp.sum(g * p, axis=1, keepdims=True)
    s = jnp.sum(p, axis=1, keepdims=True)
    mx = jnp.max(g, axis=1, keepdims=True)
    ratio = (s - pos) / (pos + EPS_)
    cls = jax.lax.broadcasted_iota(jnp.int32, (C, 1), 0)
    w = jnp.where(cls == label_ref[b], 1.0, 1.0 / C)
    contrib = jnp.where(mx != 0.0, ratio * w, 0.0)
    total = jnp.sum(contrib)
    valid = (jnp.max(mx) != 0.0).astype(jnp.float32)
    out_t_ref[0, 0, :] = jnp.full((128,), total, jnp.float32)
    out_v_ref[0, 0, :] = jnp.full((128,), valid, jnp.float32)


def _finalize_kernel(t_ref, v_ref, out_ref):
    total = jnp.sum(t_ref[:, 0, 0:1])
    n_valid = jnp.sum(v_ref[:, 0, 0:1])
    n = jnp.maximum(n_valid, 1.0)
    out_ref[0, 0] = jnp.where(total == 0.0, 0.0, jnp.log(total) / n)


@jax.jit
def kernel(Y_pred, Y_gt, label):
    B, C, H, W = Y_pred.shape
    label32 = label.astype(jnp.int32)
    Yp = Y_pred.reshape(B, C, H * W)
    Yg = Y_gt.reshape(B, C, H * W)

    out_t, out_v = pl.pallas_call(
        functools.partial(_stats_kernel, C=C),
        grid=(B,),
        in_specs=[
            pl.BlockSpec((1, C, H * W), lambda b: (b, 0, 0)),
            pl.BlockSpec((1, C, H * W), lambda b: (b, 0, 0)),
            pl.BlockSpec(memory_space=pltpu.SMEM),
        ],
        out_specs=[
            pl.BlockSpec((1, 1, 128), lambda b: (b, 0, 0)),
            pl.BlockSpec((1, 1, 128), lambda b: (b, 0, 0)),
        ],
        out_shape=[
            jax.ShapeDtypeStruct((B, 1, 128), jnp.float32),
            jax.ShapeDtypeStruct((B, 1, 128), jnp.float32),
        ],
        compiler_params=pltpu.CompilerParams(
            dimension_semantics=("parallel",),
        ),
    )(Yp, Yg, label32)

    out = pl.pallas_call(
        _finalize_kernel,
        out_specs=pl.BlockSpec(memory_space=pltpu.SMEM),
        out_shape=jax.ShapeDtypeStruct((1, 1), jnp.float32),
    )(out_t, out_v)
    return out[0, 0]


# speedup vs baseline: 1.4508x; 1.4508x over previous
import functools

import jax
import jax.numpy as jnp
from jax.experimental import pallas as pl
from jax.experimental.pallas import tpu as pltpu

EPS_ = 1e-6
_NCHUNK = 4


def _stats_kernel(*refs, C):
    p_refs = refs[:_NCHUNK]
    g_refs = refs[_NCHUNK:2 * _NCHUNK]
    label_ref = refs[2 * _NCHUNK]
    out_t_ref, out_v_ref = refs[2 * _NCHUNK + 1], refs[2 * _NCHUNK + 2]
    b = pl.program_id(0)
    lab = label_ref[b]
    total = jnp.float32(0.0)
    valid = jnp.float32(0.0)
    for c in range(C):
        pos_v = jnp.zeros((1, 128), jnp.float32)
        s_v = jnp.zeros((1, 128), jnp.float32)
        mx_v = jnp.full((1, 128), -jnp.inf, jnp.float32)
        for q in range(_NCHUNK):
            p = p_refs[q][0, c]
            g = g_refs[q][0, c]
            pos_v = pos_v + jnp.sum(g * p, axis=0, keepdims=True)
            s_v = s_v + jnp.sum(p, axis=0, keepdims=True)
            mx_v = jnp.maximum(mx_v, jnp.max(g, axis=0, keepdims=True))
        pos = jnp.sum(pos_v)
        s = jnp.sum(s_v)
        mx = jnp.max(mx_v)
        ratio = (s - pos) / (pos + EPS_)
        w = jnp.where(lab == c, 1.0, 1.0 / C)
        is_valid = mx != 0.0
        total = total + jnp.where(is_valid, ratio * w, 0.0)
        valid = jnp.maximum(valid, is_valid.astype(jnp.float32))
    out_t_ref[0, 0, :] = jnp.full((128,), total, jnp.float32)
    out_v_ref[0, 0, :] = jnp.full((128,), valid, jnp.float32)


def _finalize_kernel(t_ref, v_ref, out_ref):
    total = jnp.sum(t_ref[:, 0, 0:1])
    n_valid = jnp.sum(v_ref[:, 0, 0:1])
    n = jnp.maximum(n_valid, 1.0)
    out_ref[0, 0] = jnp.where(total == 0.0, 0.0, jnp.log(total) / n)


@jax.jit
def kernel(Y_pred, Y_gt, label):
    B, C, H, W = Y_pred.shape
    label32 = label.astype(jnp.int32)
    rows = H * W // 128
    qrows = rows // _NCHUNK
    Yp = Y_pred.reshape(B, C, rows, 128)
    Yg = Y_gt.reshape(B, C, rows, 128)

    chunk_specs = [
        pl.BlockSpec((1, C, qrows, 128), lambda b, Q=q: (b, 0, Q, 0))
        for q in range(_NCHUNK)
    ]
    out_t, out_v = pl.pallas_call(
        functools.partial(_stats_kernel, C=C),
        grid=(B,),
        in_specs=chunk_specs + chunk_specs
        + [pl.BlockSpec(memory_space=pltpu.SMEM)],
        out_specs=[
            pl.BlockSpec((1, 1, 128), lambda b: (b, 0, 0)),
            pl.BlockSpec((1, 1, 128), lambda b: (b, 0, 0)),
        ],
        out_shape=[
            jax.ShapeDtypeStruct((B, 1, 128), jnp.float32),
            jax.ShapeDtypeStruct((B, 1, 128), jnp.float32),
        ],
        compiler_params=pltpu.CompilerParams(
            dimension_semantics=("parallel",),
        ),
    )(*([Yp] * _NCHUNK), *([Yg] * _NCHUNK), label32)

    out = pl.pallas_call(
        _finalize_kernel,
        out_specs=pl.BlockSpec(memory_space=pltpu.SMEM),
        out_shape=jax.ShapeDtypeStruct((1, 1), jnp.float32),
    )(out_t, out_v)
    return out[0, 0]
